# baseline (speedup 1.0000x reference)
import functools

import jax
import jax.numpy as jnp
from jax import lax
from jax.experimental import pallas as pl
from jax.experimental.pallas import tpu as pltpu
from jax.experimental.pallas import tpu_sc as plsc

N = 10000
NFEAT = 128
NHID = 64
NCLASS = 32
K = 10
ALPHA = 0.1

NC = 2
NS = 16
CH = NCLASS // NC
NPAD = 10112
RPT = NPAD // NS
CHUNK = 512
E = 320000
NSLAB = 40
NRING = 39
EPT = NSLAB * CHUNK
E_PAD = EPT * NS


def _mlp_body(x_ref, w1_ref, b1_ref, w2_ref, b2_ref, o_ref):
    h = jnp.dot(x_ref[...], w1_ref[...], preferred_element_type=jnp.float32)
    h = jnp.maximum(h + b1_ref[...], 0.0)
    z = jnp.zeros((NPAD - N, CH), jnp.float32)
    for c in range(NC):
        o = jnp.dot(h, w2_ref[:, c * CH:(c + 1) * CH],
                    preferred_element_type=jnp.float32)
        o_ref[pl.ds(c * NPAD, N)] = o + b2_ref[:, c * CH:(c + 1) * CH]
        o_ref[pl.ds(c * NPAD + N, NPAD - N)] = z


def _mlp(x, W1, b1, W2, b2):
    return pl.pallas_call(
        _mlp_body,
        out_shape=jax.ShapeDtypeStruct((NC * NPAD, CH), jnp.float32),
    )(x, W1, b1.reshape(1, NHID), W2, b2.reshape(1, NCLASS))


def _prop_body(src_hbm, dst_hbm, ones_hbm, zeros_hbm, h_hbm, out_hbm,
               z_sh, agg_sh, src_v, dst_v, rowbuf, wsl, hz, swide,
               sqd, zbuf, semg0, semg1, semg2, sems0, sems1, sems2):
    semg = (semg0, semg1, semg2)
    sems = (sems0, sems1, sems2)
    rb = tuple(rowbuf.at[pl.ds(b * CHUNK, CHUNK)] for b in range(3))

    def zero_agg_slice(base):
        for t in range(4):
            pltpu.sync_copy(zbuf, agg_sh.at[pl.ds(base + t * (RPT // 4),
                                                  RPT // 4)])
    c = lax.axis_index("c")
    s = lax.axis_index("s")
    base = s * RPT
    hoff = c * NPAD + base

    pltpu.sync_copy(src_hbm.at[s], src_v)
    pltpu.sync_copy(dst_hbm.at[s], dst_v)
    pltpu.sync_copy(ones_hbm, rb[0])
    pltpu.sync_copy(zeros_hbm, zbuf)
    pltpu.sync_copy(h_hbm.at[pl.ds(hoff, RPT)], hz)
    zero_agg_slice(base)
    plsc.subcore_barrier()

    @pl.loop(0, NRING, step=3)
    def _deg(g):
        for u in range(3):
            j = g + u

            @pl.when(g >= 3)
            def _():
                pltpu.make_async_copy(rb[0], agg_sh.at[dst_v.at[j - 3]],
                                      sems[u]).wait()

            pltpu.async_copy(rb[0], agg_sh.at[dst_v.at[j]], sems[u],
                             add=True)

    for u in range(3):
        pltpu.make_async_copy(rb[0], agg_sh.at[dst_v.at[NRING - 3 + u]],
                              sems[u]).wait()
    pltpu.sync_copy(rb[0], agg_sh.at[dst_v.at[NSLAB - 1]], add=True)
    plsc.subcore_barrier()

    pltpu.sync_copy(agg_sh.at[pl.ds(base, RPT)], rowbuf.at[pl.ds(0, RPT)])

    @pl.loop(0, RPT, unroll=4)
    def _init(i):
        d = rowbuf[i] + 1.0
        ih = lax.bitcast_convert_type(d, jnp.int32)
        y = lax.bitcast_convert_type(0x5F3759DF - (ih >> 1), jnp.float32)
        y = y * (1.5 - 0.5 * d * y * y)
        y = y * (1.5 - 0.5 * d * y * y)
        y = y * (1.5 - 0.5 * d * y * y)
        swide[i] = (1.0 - ALPHA) * y * y
        sqd[i] = d * y
        h0 = hz[i]
        hz[i] = ALPHA * y * h0
        wsl[i] = y * h0

    zero_agg_slice(base)
    pltpu.sync_copy(wsl, z_sh.at[pl.ds(base, RPT)])
    plsc.subcore_barrier()

    @pl.loop(0, K)
    def _hop(k):
        pltpu.async_copy(z_sh.at[src_v.at[0]], rb[0], semg[0])
        pltpu.async_copy(z_sh.at[src_v.at[1]], rb[1], semg[1])

        @pl.loop(0, NRING, step=3)
        def _edges(g):
            for u in range(3):
                j = g + u
                b = u
                bn = (u + 2) % 3
                pltpu.make_async_copy(z_sh.at[src_v.at[j]], rb[b],
                                      semg[b]).wait()
                pltpu.async_copy(rb[b], agg_sh.at[dst_v.at[j]], sems[b],
                                 add=True)
                if u == 0:
                    @pl.when(g > 0)
                    def _():
                        pltpu.make_async_copy(
                            rb[bn], agg_sh.at[dst_v.at[j - 1]],
                            sems[bn]).wait()

                    pltpu.async_copy(z_sh.at[src_v.at[j + 2]], rb[bn],
                                     semg[bn])
                else:
                    @pl.when(j + 2 < NRING + 1)
                    def _():
                        pltpu.make_async_copy(
                            rb[bn], agg_sh.at[dst_v.at[j - 1]],
                            sems[bn]).wait()
                        pltpu.async_copy(z_sh.at[src_v.at[j + 2]], rb[bn],
                                         semg[bn])

        for j in (NRING - 2, NRING - 1):
            pltpu.make_async_copy(rb[j % 3], agg_sh.at[dst_v.at[j]],
                                  sems[j % 3]).wait()
        pltpu.make_async_copy(z_sh.at[src_v.at[NSLAB - 1]], rb[0],
                              semg[0]).wait()
        pltpu.sync_copy(rb[0], agg_sh.at[dst_v.at[NSLAB - 1]], add=True)

        plsc.subcore_barrier()
        pltpu.sync_copy(agg_sh.at[pl.ds(base, RPT)], rowbuf.at[pl.ds(0, RPT)])
        zero_agg_slice(base)

        @pl.loop(0, RPT, unroll=8)
        def _comb(i):
            wsl[i] = swide[i] * (rowbuf[i] + wsl[i]) + hz[i]

        pltpu.sync_copy(wsl, z_sh.at[pl.ds(base, RPT)])
        plsc.subcore_barrier()

    @pl.loop(0, RPT, unroll=8)
    def _fin(i):
        wsl[i] = sqd[i] * wsl[i]

    pltpu.sync_copy(wsl, out_hbm.at[pl.ds(hoff, RPT)])


_prop = functools.partial(
    pl.kernel,
    _prop_body,
    out_type=jax.ShapeDtypeStruct((2 * NPAD, CH), jnp.float32),
    mesh=plsc.VectorSubcoreMesh(
        core_axis_name="c", subcore_axis_name="s", num_cores=NC,
        num_subcores=NS),
    compiler_params=pltpu.CompilerParams(use_tc_tiling_on_sc=False),
    scratch_types=[
        pltpu.VMEM_SHARED((NPAD, CH), jnp.float32),
        pltpu.VMEM_SHARED((NPAD, CH), jnp.float32),
        pltpu.VMEM((NSLAB, CHUNK), jnp.int32),
        pltpu.VMEM((NSLAB, CHUNK), jnp.int32),
        pltpu.VMEM((3 * CHUNK, CH), jnp.float32),
        pltpu.VMEM((RPT, CH), jnp.float32),
        pltpu.VMEM((RPT, CH), jnp.float32),
        pltpu.VMEM((RPT, CH), jnp.float32),
        pltpu.VMEM((RPT, CH), jnp.float32),
        pltpu.VMEM((RPT // 4, CH), jnp.float32),
        pltpu.SemaphoreType.DMA,
        pltpu.SemaphoreType.DMA,
        pltpu.SemaphoreType.DMA,
        pltpu.SemaphoreType.DMA,
        pltpu.SemaphoreType.DMA,
        pltpu.SemaphoreType.DMA,
    ],
)()


def kernel(x, edge_index, W1, b1, W2, b2):
    h2 = _mlp(x, W1, b1, W2, b2)

    pad = N + jnp.arange(E_PAD - E, dtype=jnp.int32) % (NPAD - N)
    src = jnp.concatenate([edge_index[0], pad])
    dst = jnp.concatenate([edge_index[1], pad])
    src3 = src.reshape(NS, NSLAB, CHUNK)
    dst3 = dst.reshape(NS, NSLAB, CHUNK)

    ones_rows = jnp.ones((CHUNK, CH), jnp.float32)
    zero_rows = jnp.zeros((RPT // 4, CH), jnp.float32)

    out2 = _prop(src3, dst3, ones_rows, zero_rows, h2)
    out = out2.reshape(NC, NPAD, CH)[:, :N, :].transpose(1, 0, 2)
    return out.reshape(N, NCLASS)


# speedup vs baseline: 1.0381x; 1.0381x over previous
import functools

import jax
import jax.numpy as jnp
from jax import lax
from jax.experimental import pallas as pl
from jax.experimental.pallas import tpu as pltpu
from jax.experimental.pallas import tpu_sc as plsc

N = 10000
NFEAT = 128
NHID = 64
NCLASS = 32
K = 10
ALPHA = 0.1

NC = 2
NS = 16
CH = NCLASS // NC
NPAD = 10112
RPT = NPAD // NS
CHUNK = 512
E = 320000
NSLAB = 40
NRING = 39
EPT = NSLAB * CHUNK
E_PAD = EPT * NS


def _mlp_body(x_ref, w1_ref, b1_ref, w2_ref, b2_ref, o_ref):
    h = jnp.dot(x_ref[...], w1_ref[...], preferred_element_type=jnp.float32)
    h = jnp.maximum(h + b1_ref[...], 0.0)
    z = jnp.zeros((NPAD - N, CH), jnp.float32)
    for c in range(NC):
        o = jnp.dot(h, w2_ref[:, c * CH:(c + 1) * CH],
                    preferred_element_type=jnp.float32)
        o_ref[pl.ds(c * NPAD, N)] = o + b2_ref[:, c * CH:(c + 1) * CH]
        o_ref[pl.ds(c * NPAD + N, NPAD - N)] = z


def _mlp(x, W1, b1, W2, b2):
    return pl.pallas_call(
        _mlp_body,
        out_shape=jax.ShapeDtypeStruct((NC * NPAD, CH), jnp.float32),
    )(x, W1, b1.reshape(1, NHID), W2, b2.reshape(1, NCLASS))


def _prop_body(src_hbm, dst_hbm, ones_hbm, zeros_hbm, h_hbm, out_hbm,
               z_sh, agg_sh, src_v, dst_v, rowbuf, wsl, hz, swide,
               sqd, zbuf, semg0, semg1, semg2, sems0, sems1, sems2):
    semg = (semg0, semg1, semg2)
    sems = (sems0, sems1, sems2)
    rb = tuple(rowbuf.at[pl.ds(b * CHUNK, CHUNK)] for b in range(3))

    def zero_agg_slice(base):
        for t in range(4):
            pltpu.sync_copy(zbuf, agg_sh.at[pl.ds(base + t * (RPT // 4),
                                                  RPT // 4)])
    c = lax.axis_index("c")
    s = lax.axis_index("s")
    base = s * RPT
    hoff = c * NPAD + base

    pltpu.sync_copy(src_hbm.at[s], src_v)
    pltpu.sync_copy(dst_hbm.at[s], dst_v)
    pltpu.sync_copy(ones_hbm, rb[0])
    pltpu.sync_copy(zeros_hbm, zbuf)
    pltpu.sync_copy(h_hbm.at[pl.ds(hoff, RPT)], hz)
    zero_agg_slice(base)
    plsc.subcore_barrier()

    @pl.loop(0, NRING, step=3)
    def _deg(g):
        for u in range(3):
            j = g + u

            @pl.when(g >= 3)
            def _():
                pltpu.make_async_copy(rb[0], agg_sh.at[dst_v.at[j - 3]],
                                      sems[u]).wait()

            pltpu.async_copy(rb[0], agg_sh.at[dst_v.at[j]], sems[u],
                             add=True)

    for u in range(3):
        pltpu.make_async_copy(rb[0], agg_sh.at[dst_v.at[NRING - 3 + u]],
                              sems[u]).wait()
    pltpu.sync_copy(rb[0], agg_sh.at[dst_v.at[NSLAB - 1]], add=True)
    plsc.subcore_barrier()

    pltpu.sync_copy(agg_sh.at[pl.ds(base, RPT)], rowbuf.at[pl.ds(0, RPT)])

    @pl.loop(0, RPT)
    def _init(i):
        d = rowbuf[i] + 1.0
        ih = lax.bitcast_convert_type(d, jnp.int32)
        y = lax.bitcast_convert_type(0x5F3759DF - (ih >> 1), jnp.float32)
        y = y * (1.5 - 0.5 * d * y * y)
        y = y * (1.5 - 0.5 * d * y * y)
        y = y * (1.5 - 0.5 * d * y * y)
        swide[i] = (1.0 - ALPHA) * y * y
        sqd[i] = d * y
        h0 = hz[i]
        hz[i] = ALPHA * y * h0
        wsl[i] = y * h0

    zero_agg_slice(base)
    pltpu.sync_copy(wsl, z_sh.at[pl.ds(base, RPT)])
    plsc.subcore_barrier()

    @pl.loop(0, K)
    def _hop(k):
        pltpu.async_copy(z_sh.at[src_v.at[0]], rb[0], semg[0])
        pltpu.async_copy(z_sh.at[src_v.at[1]], rb[1], semg[1])

        @pl.loop(0, NRING, step=3)
        def _edges(g):
            for u in range(3):
                j = g + u
                b = u
                bn = (u + 2) % 3
                pltpu.make_async_copy(z_sh.at[src_v.at[j]], rb[b],
                                      semg[b]).wait()
                pltpu.async_copy(rb[b], agg_sh.at[dst_v.at[j]], sems[b],
                                 add=True)
                if u == 0:
                    @pl.when(g > 0)
                    def _():
                        pltpu.make_async_copy(
                            rb[bn], agg_sh.at[dst_v.at[j - 1]],
                            sems[bn]).wait()

                    pltpu.async_copy(z_sh.at[src_v.at[j + 2]], rb[bn],
                                     semg[bn])
                else:
                    @pl.when(j + 2 < NRING + 1)
                    def _():
                        pltpu.make_async_copy(
                            rb[bn], agg_sh.at[dst_v.at[j - 1]],
                            sems[bn]).wait()
                        pltpu.async_copy(z_sh.at[src_v.at[j + 2]], rb[bn],
                                         semg[bn])

        for j in (NRING - 2, NRING - 1):
            pltpu.make_async_copy(rb[j % 3], agg_sh.at[dst_v.at[j]],
                                  sems[j % 3]).wait()
        pltpu.make_async_copy(z_sh.at[src_v.at[NSLAB - 1]], rb[0],
                              semg[0]).wait()
        pltpu.sync_copy(rb[0], agg_sh.at[dst_v.at[NSLAB - 1]], add=True)

        plsc.subcore_barrier()
        pltpu.sync_copy(agg_sh.at[pl.ds(base, RPT)], rowbuf.at[pl.ds(0, RPT)])
        zero_agg_slice(base)

        @pl.loop(0, RPT)
        def _comb(i):
            wsl[i] = swide[i] * (rowbuf[i] + wsl[i]) + hz[i]

        pltpu.sync_copy(wsl, z_sh.at[pl.ds(base, RPT)])
        plsc.subcore_barrier()

    @pl.loop(0, RPT)
    def _fin(i):
        wsl[i] = sqd[i] * wsl[i]

    pltpu.sync_copy(wsl, out_hbm.at[pl.ds(hoff, RPT)])


_prop = functools.partial(
    pl.kernel,
    _prop_body,
    out_type=jax.ShapeDtypeStruct((2 * NPAD, CH), jnp.float32),
    mesh=plsc.VectorSubcoreMesh(
        core_axis_name="c", subcore_axis_name="s", num_cores=NC,
        num_subcores=NS),
    compiler_params=pltpu.CompilerParams(use_tc_tiling_on_sc=False),
    scratch_types=[
        pltpu.VMEM_SHARED((NPAD, CH), jnp.float32),
        pltpu.VMEM_SHARED((NPAD, CH), jnp.float32),
        pltpu.VMEM((NSLAB, CHUNK), jnp.int32),
        pltpu.VMEM((NSLAB, CHUNK), jnp.int32),
        pltpu.VMEM((3 * CHUNK, CH), jnp.float32),
        pltpu.VMEM((RPT, CH), jnp.float32),
        pltpu.VMEM((RPT, CH), jnp.float32),
        pltpu.VMEM((RPT, CH), jnp.float32),
        pltpu.VMEM((RPT, CH), jnp.float32),
        pltpu.VMEM((RPT // 4, CH), jnp.float32),
        pltpu.SemaphoreType.DMA,
        pltpu.SemaphoreType.DMA,
        pltpu.SemaphoreType.DMA,
        pltpu.SemaphoreType.DMA,
        pltpu.SemaphoreType.DMA,
        pltpu.SemaphoreType.DMA,
    ],
)()


def kernel(x, edge_index, W1, b1, W2, b2):
    h2 = _mlp(x, W1, b1, W2, b2)

    pad = N + jnp.arange(E_PAD - E, dtype=jnp.int32) % (NPAD - N)
    src = jnp.concatenate([edge_index[0], pad])
    dst = jnp.concatenate([edge_index[1], pad])
    src3 = src.reshape(NS, NSLAB, CHUNK)
    dst3 = dst.reshape(NS, NSLAB, CHUNK)

    ones_rows = jnp.ones((CHUNK, CH), jnp.float32)
    zero_rows = jnp.zeros((RPT // 4, CH), jnp.float32)

    out2 = _prop(src3, dst3, ones_rows, zero_rows, h2)
    out = out2.reshape(NC, NPAD, CH)[:, :N, :].transpose(1, 0, 2)
    return out.reshape(N, NCLASS)


# speedup vs baseline: 1.0773x; 1.0378x over previous
import functools

import jax
import jax.numpy as jnp
from jax import lax
from jax.experimental import pallas as pl
from jax.experimental.pallas import tpu as pltpu
from jax.experimental.pallas import tpu_sc as plsc

N = 10000
NFEAT = 128
NHID = 64
NCLASS = 32
K = 10
ALPHA = 0.1

NC = 2
NS = 16
CH = NCLASS // NC
NPAD = 10112
RPT = NPAD // NS
CHUNK = 512
E = 320000
NSLAB = 40
NRING = 39
EPT = NSLAB * CHUNK
E_PAD = EPT * NS


def _mlp_body(x_ref, w1_ref, b1_ref, w2_ref, b2_ref, o_ref):
    h = jnp.dot(x_ref[...], w1_ref[...], preferred_element_type=jnp.float32)
    h = jnp.maximum(h + b1_ref[...], 0.0)
    z = jnp.zeros((NPAD - N, CH), jnp.float32)
    for c in range(NC):
        o = jnp.dot(h, w2_ref[:, c * CH:(c + 1) * CH],
                    preferred_element_type=jnp.float32)
        o_ref[pl.ds(c * NPAD, N)] = o + b2_ref[:, c * CH:(c + 1) * CH]
        o_ref[pl.ds(c * NPAD + N, NPAD - N)] = z


def _mlp(x, W1, b1, W2, b2):
    return pl.pallas_call(
        _mlp_body,
        out_shape=jax.ShapeDtypeStruct((NC * NPAD, CH), jnp.float32),
    )(x, W1, b1.reshape(1, NHID), W2, b2.reshape(1, NCLASS))


def _prop_body(src_hbm, dst_hbm, ones_hbm, zeros_hbm, h_hbm, out_hbm,
               z_sh, agg_sh, src_v, dst_v, rowbuf, wsl, hz, swide,
               sqd, zbuf, semg0, semg1, semg2, sems0, sems1, sems2):
    semg = (semg0, semg1, semg2)
    sems = (sems0, sems1, sems2)
    rb = tuple(rowbuf.at[pl.ds(b * CHUNK, CHUNK)] for b in range(3))

    def zero_agg_slice(base):
        for t in range(4):
            pltpu.sync_copy(zbuf, agg_sh.at[pl.ds(base + t * (RPT // 4),
                                                  RPT // 4)])
    c = lax.axis_index("c")
    s = lax.axis_index("s")
    base = s * RPT
    hoff = c * NPAD + base

    pltpu.sync_copy(src_hbm.at[s], src_v)
    pltpu.sync_copy(dst_hbm.at[s], dst_v)
    pltpu.sync_copy(ones_hbm, rb[0])
    pltpu.sync_copy(zeros_hbm, zbuf)
    pltpu.sync_copy(h_hbm.at[pl.ds(hoff, RPT)], hz)
    zero_agg_slice(base)
    plsc.subcore_barrier()

    @pl.loop(0, NRING, step=3)
    def _deg(g):
        for u in range(3):
            j = g + u

            @pl.when(g >= 3)
            def _():
                pltpu.make_async_copy(rb[0], agg_sh.at[dst_v.at[j - 3]],
                                      sems[u]).wait()

            pltpu.async_copy(rb[0], agg_sh.at[dst_v.at[j]], sems[u],
                             add=True)

    for u in range(3):
        pltpu.make_async_copy(rb[0], agg_sh.at[dst_v.at[NRING - 3 + u]],
                              sems[u]).wait()
    pltpu.sync_copy(rb[0], agg_sh.at[dst_v.at[NSLAB - 1]], add=True)
    plsc.subcore_barrier()

    pltpu.sync_copy(agg_sh.at[pl.ds(base, RPT)], rowbuf.at[pl.ds(0, RPT)])

    @pl.loop(0, RPT)
    def _init(i):
        d = rowbuf[i] + 1.0
        ih = lax.bitcast_convert_type(d, jnp.int32)
        y = lax.bitcast_convert_type(0x5F3759DF - (ih >> 1), jnp.float32)
        y = y * (1.5 - 0.5 * d * y * y)
        y = y * (1.5 - 0.5 * d * y * y)
        y = y * (1.5 - 0.5 * d * y * y)
        swide[i] = (1.0 - ALPHA) * y * y
        sqd[i] = d * y
        h0 = hz[i]
        hz[i] = ALPHA * y * h0
        wsl[i] = y * h0

    zero_agg_slice(base)
    pltpu.sync_copy(wsl, z_sh.at[pl.ds(base, RPT)])
    plsc.subcore_barrier()

    @pl.loop(0, K)
    def _hop(k):
        pltpu.async_copy(z_sh.at[src_v.at[0]], rb[0], semg[0])
        pltpu.async_copy(z_sh.at[src_v.at[1]], rb[1], semg[1])

        @pl.loop(0, NRING, step=3)
        def _edges(g):
            for u in range(3):
                j = g + u
                b = u
                bn = (u + 2) % 3
                pltpu.make_async_copy(z_sh.at[src_v.at[j]], rb[b],
                                      semg[b]).wait()
                pltpu.async_copy(rb[b], agg_sh.at[dst_v.at[j]], sems[b],
                                 add=True)
                if u == 0:
                    @pl.when(g > 0)
                    def _():
                        pltpu.make_async_copy(
                            rb[bn], agg_sh.at[dst_v.at[j - 1]],
                            sems[bn]).wait()

                    pltpu.async_copy(z_sh.at[src_v.at[j + 2]], rb[bn],
                                     semg[bn])
                else:
                    @pl.when(j + 2 < NRING + 1)
                    def _():
                        pltpu.make_async_copy(
                            rb[bn], agg_sh.at[dst_v.at[j - 1]],
                            sems[bn]).wait()
                        pltpu.async_copy(z_sh.at[src_v.at[j + 2]], rb[bn],
                                         semg[bn])

        for j in (NRING - 2, NRING - 1):
            pltpu.make_async_copy(rb[j % 3], agg_sh.at[dst_v.at[j]],
                                  sems[j % 3]).wait()
        pltpu.make_async_copy(z_sh.at[src_v.at[NSLAB - 1]], rb[0],
                              semg[0]).wait()
        pltpu.sync_copy(rb[0], agg_sh.at[dst_v.at[NSLAB - 1]], add=True)

        plsc.subcore_barrier()
        pltpu.sync_copy(agg_sh.at[pl.ds(base, RPT)], rowbuf.at[pl.ds(0, RPT)])
        zero_agg_slice(base)

        @pl.loop(0, RPT)
        def _comb(i):
            wsl[i] = swide[i] * (rowbuf[i] + wsl[i]) + hz[i]

        pltpu.sync_copy(wsl, z_sh.at[pl.ds(base, RPT)])
        plsc.subcore_barrier()

    @pl.loop(0, RPT)
    def _fin(i):
        wsl[i] = sqd[i] * wsl[i]

    col = c * CH
    @pl.when(s < NS - 1)
    def _():
        pltpu.sync_copy(wsl, out_hbm.at[pl.ds(base, RPT), pl.ds(col, CH)])

    @pl.when(s == NS - 1)
    def _():
        pltpu.sync_copy(wsl.at[pl.ds(0, N - (NS - 1) * RPT)],
                        out_hbm.at[pl.ds(base, N - (NS - 1) * RPT),
                                   pl.ds(col, CH)])


_prop = functools.partial(
    pl.kernel,
    _prop_body,
    out_type=jax.ShapeDtypeStruct((N, NCLASS), jnp.float32),
    mesh=plsc.VectorSubcoreMesh(
        core_axis_name="c", subcore_axis_name="s", num_cores=NC,
        num_subcores=NS),
    compiler_params=pltpu.CompilerParams(use_tc_tiling_on_sc=False),
    scratch_types=[
        pltpu.VMEM_SHARED((NPAD, CH), jnp.float32),
        pltpu.VMEM_SHARED((NPAD, CH), jnp.float32),
        pltpu.VMEM((NSLAB, CHUNK), jnp.int32),
        pltpu.VMEM((NSLAB, CHUNK), jnp.int32),
        pltpu.VMEM((3 * CHUNK, CH), jnp.float32),
        pltpu.VMEM((RPT, CH), jnp.float32),
        pltpu.VMEM((RPT, CH), jnp.float32),
        pltpu.VMEM((RPT, CH), jnp.float32),
        pltpu.VMEM((RPT, CH), jnp.float32),
        pltpu.VMEM((RPT // 4, CH), jnp.float32),
        pltpu.SemaphoreType.DMA,
        pltpu.SemaphoreType.DMA,
        pltpu.SemaphoreType.DMA,
        pltpu.SemaphoreType.DMA,
        pltpu.SemaphoreType.DMA,
        pltpu.SemaphoreType.DMA,
    ],
)()


def kernel(x, edge_index, W1, b1, W2, b2):
    h2 = _mlp(x, W1, b1, W2, b2)

    pad = N + jnp.arange(E_PAD - E, dtype=jnp.int32) % (NPAD - N)
    src = jnp.concatenate([edge_index[0], pad])
    dst = jnp.concatenate([edge_index[1], pad])
    src3 = src.reshape(NS, NSLAB, CHUNK)
    dst3 = dst.reshape(NS, NSLAB, CHUNK)

    ones_rows = jnp.ones((CHUNK, CH), jnp.float32)
    zero_rows = jnp.zeros((RPT // 4, CH), jnp.float32)

    return _prop(src3, dst3, ones_rows, zero_rows, h2)
